# Initial kernel scaffold; baseline (speedup 1.0000x reference)
import jax
import jax.numpy as jnp
from jax.experimental import pallas as pl


def kernel(cloud_a, cloud_b, W, b, gamma, beta):
    raise NotImplementedError("write your pallas kernel here")


# speedup vs baseline: 4.7125x; 4.7125x over previous
import jax
import jax.numpy as jnp
from jax.experimental import pallas as pl

_N = 2048
_K = 20
_R = 256
_C = 3


def _knn_feat_kernel(xq_ref, xf_ref, w_ref, b_ref, y_ref, s1_ref, s2_ref):
    xq = xq_ref[0]
    xf = xf_ref[0]
    v = [xq[:, c:c + 1] for c in range(_C)]
    xk = [xf[c:c + 1, :] for c in range(_C)]

    xxq = v[0] * v[0] + v[1] * v[1] + v[2] * v[2]
    xxf = xk[0] * xk[0] + xk[1] * xk[1] + xk[2] * xk[2]
    s = jax.lax.dot_general(xq, xf, (((1,), (0,)), ((), ())),
                            preferred_element_type=jnp.float32)
    inner = -2.0 * s
    dist = (-xxq - inner) - xxf

    iota = jax.lax.broadcasted_iota(jnp.int32, (_R, _N), 1)
    ycols = [[] for _ in range(6)]
    for t in range(_K):
        m = jnp.max(dist, axis=1, keepdims=True)
        ii = jnp.where(dist >= m, iota, _N)
        idx = jnp.min(ii, axis=1, keepdims=True)
        sel = iota == idx
        u = [jnp.sum(jnp.where(sel, xk[c], 0.0), axis=1, keepdims=True)
             for c in range(_C)]
        if t < _K - 1:
            dist = jnp.where(sel, -jnp.inf, dist)
        f = [u[c] - v[c] for c in range(_C)] + v
        for o in range(6):
            y_o = f[0] * w_ref[o:o + 1, 0:1]
            for i in range(1, 6):
                y_o = y_o + f[i] * w_ref[o:o + 1, i:i + 1]
            y_o = y_o + b_ref[0:1, o:o + 1]
            ycols[o].append(y_o)

    s1_rows = []
    s2_rows = []
    for o in range(6):
        yk = jnp.concatenate(ycols[o], axis=1)
        y_ref[0, o] = yk
        s1 = jnp.sum(yk, axis=(0, 1), keepdims=True)
        s2 = jnp.sum(yk * yk, axis=(0, 1), keepdims=True)
        s1_rows.append(jnp.broadcast_to(s1, (1, 128)))
        s2_rows.append(jnp.broadcast_to(s2, (1, 128)))
    s1_ref[0, 0] = jnp.concatenate(s1_rows, axis=0)
    s2_ref[0, 0] = jnp.concatenate(s2_rows, axis=0)


def _bn_lrelu_kernel(y_ref, a_ref, c_ref, o_ref):
    for o in range(6):
        t = y_ref[0, o] * a_ref[0, 0:1, o:o + 1] + c_ref[0, 0:1, o:o + 1]
        o_ref[0, o] = jnp.where(t >= 0, t, 0.2 * t)


def kernel(cloud_a, cloud_b, W, b, gamma, beta):
    eps = 1e-5
    X = jnp.concatenate([cloud_a, cloud_b], axis=0)
    Xt = jnp.transpose(X, (0, 2, 1))
    b2 = b.reshape(1, 6)
    nb = X.shape[0]
    nrb = _N // _R

    y, s1, s2 = pl.pallas_call(
        _knn_feat_kernel,
        grid=(nb, nrb),
        in_specs=[
            pl.BlockSpec((1, _R, _C), lambda i, j: (i, j, 0)),
            pl.BlockSpec((1, _C, _N), lambda i, j: (i, 0, 0)),
            pl.BlockSpec((6, 6), lambda i, j: (0, 0)),
            pl.BlockSpec((1, 6), lambda i, j: (0, 0)),
        ],
        out_specs=[
            pl.BlockSpec((1, 6, _R, _K), lambda i, j: (i, 0, j, 0)),
            pl.BlockSpec((1, 1, 6, 128), lambda i, j: (i, j, 0, 0)),
            pl.BlockSpec((1, 1, 6, 128), lambda i, j: (i, j, 0, 0)),
        ],
        out_shape=[
            jax.ShapeDtypeStruct((nb, 6, _N, _K), jnp.float32),
            jax.ShapeDtypeStruct((nb, nrb, 6, 128), jnp.float32),
            jax.ShapeDtypeStruct((nb, nrb, 6, 128), jnp.float32),
        ],
    )(Xt, X, W, b2)

    cnt = jnp.float32(8 * _N * _K)
    sy = s1[:, :, :, 0].sum(axis=1).reshape(2, 8, 6).sum(axis=1)
    sy2 = s2[:, :, :, 0].sum(axis=1).reshape(2, 8, 6).sum(axis=1)
    mean = sy / cnt
    var = sy2 / cnt - mean * mean
    inv = gamma[None, :] / jnp.sqrt(var + eps)
    A = jnp.repeat(inv, 8, axis=0).reshape(16, 1, 6)
    Bc = jnp.repeat(beta[None, :] - mean * inv, 8, axis=0).reshape(16, 1, 6)

    out = pl.pallas_call(
        _bn_lrelu_kernel,
        grid=(nb,),
        in_specs=[
            pl.BlockSpec((1, 6, _N, _K), lambda i: (i, 0, 0, 0)),
            pl.BlockSpec((1, 1, 6), lambda i: (i, 0, 0)),
            pl.BlockSpec((1, 1, 6), lambda i: (i, 0, 0)),
        ],
        out_specs=pl.BlockSpec((1, 6, _N, _K), lambda i: (i, 0, 0, 0)),
        out_shape=jax.ShapeDtypeStruct((nb, 6, _N, _K), jnp.float32),
    )(y, A, Bc)

    return out[:8], out[8:]
